# baseline (speedup 1.0000x reference)
import functools

import jax
import jax.numpy as jnp
from jax import lax
from jax.experimental import pallas as pl
from jax.experimental.pallas import tpu as pltpu
from jax.experimental.pallas import tpu_sc as plsc

V = 100000
D = 128
L_SEQ = 200
B = 4096
_INV_L = 1.0 / L_SEQ

_QROWS = 256
_PBLK = _QROWS * D
_GRID = (V + _PBLK - 1) // _PBLK
_QR_TOTAL = _GRID * _QROWS


def _proj_body(b_ref, t_ref, w_ref, q_ref):
    t = t_ref[...]
    w = w_ref[...]
    s = (t * w).reshape(_QROWS, D, D)
    q_ref[...] = jnp.sum(s, axis=2) * _INV_L + b_ref[0] * _INV_L


def _project_table(table, W, b):
    q2 = pl.pallas_call(
        _proj_body,
        grid=(_GRID,),
        in_specs=[
            pl.BlockSpec(memory_space=pltpu.SMEM),
            pl.BlockSpec((_PBLK, D), lambda i: (i, 0)),
            pl.BlockSpec((1, D), lambda i: (0, 0)),
        ],
        out_specs=pl.BlockSpec((_QROWS, D), lambda i: (i, 0)),
        out_shape=jax.ShapeDtypeStruct((_QR_TOTAL, D), jnp.float32),
    )(b, table, W)
    return q2.reshape(_QR_TOTAL * D)


_NC, _NS = 2, 16
_NW = _NC * _NS
_BPW = B // _NW
_G = _BPW // 16


@functools.cache
def _sc_gather_sum_fn():
    mesh = plsc.VectorSubcoreMesh(core_axis_name="c", subcore_axis_name="s")

    @functools.partial(
        pl.kernel,
        mesh=mesh,
        compiler_params=pltpu.CompilerParams(needs_layout_passes=False),
        out_type=jax.ShapeDtypeStruct((B,), jnp.float32),
        scratch_types=[
            pltpu.VMEM((V,), jnp.float32),
            pltpu.VMEM((L_SEQ, _BPW), jnp.int32),
            pltpu.VMEM((_BPW,), jnp.float32),
            pltpu.SemaphoreType.DMA,
            pltpu.SemaphoreType.DMA,
            pltpu.SemaphoreType.DMA,
            pltpu.SemaphoreType.DMA,
            pltpu.SemaphoreType.DMA,
            pltpu.SemaphoreType.DMA,
        ],
    )
    def _sc_gather_sum(q_hbm, x_hbm, out_hbm, q_v, x_v, acc_v,
                       sq0, sq1, sq2, sq3, sx0, sx1):
        wid = lax.axis_index("s") * _NC + lax.axis_index("c")
        base = wid * _BPW
        qw = V // 4
        copies = []
        for k, sem in enumerate((sq0, sq1, sq2, sq3)):
            copies.append(pltpu.async_copy(
                q_hbm.at[pl.ds(k * qw, qw)], q_v.at[pl.ds(k * qw, qw)], sem))
        for r0, nr, sem in ((0, 104, sx0), (104, 96, sx1)):
            copies.append(pltpu.async_copy(
                x_hbm.at[pl.ds(r0, nr), pl.ds(base, _BPW)],
                x_v.at[pl.ds(r0, nr), :], sem))
        for c in copies:
            c.wait()

        def body(l, accs):
            new = []
            for g in range(_G):
                idx = x_v[l, pl.ds(g * 16, 16)]
                new.append(accs[g] + plsc.load_gather(q_v, [idx]))
            return tuple(new)

        accs = lax.fori_loop(
            0, L_SEQ, body,
            tuple(jnp.zeros((16,), jnp.float32) for _ in range(_G)))
        for g in range(_G):
            acc_v[pl.ds(g * 16, 16)] = accs[g]
        pltpu.sync_copy(acc_v, out_hbm.at[pl.ds(base, _BPW)])

    return _sc_gather_sum


def kernel(x, lens, table, W, b):
    del lens
    q = _project_table(table, W, b)
    return _sc_gather_sum_fn()(q, x)


# speedup vs baseline: 1.0623x; 1.0623x over previous
import functools

import jax
import jax.numpy as jnp
from jax import lax
from jax.experimental import pallas as pl
from jax.experimental.pallas import tpu as pltpu
from jax.experimental.pallas import tpu_sc as plsc

V = 100000
D = 128
L_SEQ = 200
B = 4096
_INV_L = 1.0 / L_SEQ

_QROWS = 128
_PBLK = _QROWS * D
_GRID = (V + _PBLK - 1) // _PBLK
_QPROWS = _QROWS // 2
_QP_TOTAL = _GRID * _QPROWS


def _rne_bf16_bits(q):
    u = lax.bitcast_convert_type(q, jnp.uint32)
    lsb = (u >> 16) & jnp.uint32(1)
    return (u + jnp.uint32(0x7FFF) + lsb) >> 16


def _proj_body(b_ref, t_ref, w_ref, qp_ref):
    t = t_ref[...]
    w = w_ref[...]
    s = (t * w).reshape(_QROWS, D, D)
    q = jnp.sum(s, axis=2) * _INV_L + b_ref[0] * _INV_L
    q3 = q.reshape(_QPROWS, 2, D)
    lo = _rne_bf16_bits(q3[:, 0, :])
    hi = _rne_bf16_bits(q3[:, 1, :])
    qp_ref[...] = lo | (hi << 16)


def _project_table(table, W, b):
    qp = pl.pallas_call(
        _proj_body,
        grid=(_GRID,),
        in_specs=[
            pl.BlockSpec(memory_space=pltpu.SMEM),
            pl.BlockSpec((_PBLK, D), lambda i: (i, 0)),
            pl.BlockSpec((1, D), lambda i: (0, 0)),
        ],
        out_specs=pl.BlockSpec((_QPROWS, D), lambda i: (i, 0)),
        out_shape=jax.ShapeDtypeStruct((_QP_TOTAL, D), jnp.uint32),
    )(b, table, W)
    return qp.reshape(_QP_TOTAL * D)


_NC, _NS = 2, 16
_NW = _NC * _NS
_BPW = B // _NW
_G = _BPW // 16
_QPW = _QP_TOTAL * D


@functools.cache
def _sc_gather_sum_fn():
    mesh = plsc.VectorSubcoreMesh(core_axis_name="c", subcore_axis_name="s")

    @functools.partial(
        pl.kernel,
        mesh=mesh,
        compiler_params=pltpu.CompilerParams(needs_layout_passes=False),
        out_type=jax.ShapeDtypeStruct((B,), jnp.float32),
        scratch_types=[
            pltpu.VMEM((_QPW,), jnp.int32),
            pltpu.VMEM((L_SEQ, _BPW), jnp.int32),
            pltpu.VMEM((_BPW,), jnp.float32),
            pltpu.SemaphoreType.DMA,
            pltpu.SemaphoreType.DMA,
            pltpu.SemaphoreType.DMA,
            pltpu.SemaphoreType.DMA,
        ],
    )
    def _sc_gather_sum(qp_hbm, x_hbm, out_hbm, q_v, x_v, acc_v,
                       sq0, sq1, sx0, sx1):
        wid = lax.axis_index("s") * _NC + lax.axis_index("c")
        base = wid * _BPW
        qw = _QPW // 2
        copies = []
        for k, sem in enumerate((sq0, sq1)):
            copies.append(pltpu.async_copy(
                qp_hbm.at[pl.ds(k * qw, qw)], q_v.at[pl.ds(k * qw, qw)], sem))
        for r0, nr, sem in ((0, 104, sx0), (104, 96, sx1)):
            copies.append(pltpu.async_copy(
                x_hbm.at[pl.ds(r0, nr), pl.ds(base, _BPW)],
                x_v.at[pl.ds(r0, nr), :], sem))
        for c in copies:
            c.wait()

        def body(l, accs):
            new = []
            for g in range(_G):
                v = x_v[l, pl.ds(g * 16, 16)]
                wi = ((v >> 1) & jnp.int32(-128)) | (v & 127)
                word = plsc.load_gather(q_v, [wi])
                shift = (v >> 3) & 16
                bits = lax.shift_right_logical(word, shift) << 16
                val = plsc.bitcast(bits, jnp.float32)
                new.append(accs[g] + val)
            return tuple(new)

        accs = lax.fori_loop(
            0, L_SEQ, body,
            tuple(jnp.zeros((16,), jnp.float32) for _ in range(_G)))
        for g in range(_G):
            acc_v[pl.ds(g * 16, 16)] = accs[g]
        pltpu.sync_copy(acc_v, out_hbm.at[pl.ds(base, _BPW)])

    return _sc_gather_sum


def kernel(x, lens, table, W, b):
    del lens
    qp = _project_table(table, W, b)
    return _sc_gather_sum_fn()(qp, x)
